# baseline (speedup 1.0000x reference)
import functools

import jax
import jax.numpy as jnp
from jax import lax
from jax.experimental import pallas as pl
from jax.experimental.pallas import tpu as pltpu
from jax.experimental.pallas import tpu_sc as plsc

_L = 16


def _pos_embed_sc(row_embed, col_embed, B, H, W, D):
    C = 2 * D
    NW = 32
    ROWS = C // NW
    NROW, DROW = row_embed.shape
    NCOL, DCOL = col_embed.shape
    mesh = plsc.VectorSubcoreMesh(core_axis_name="c", subcore_axis_name="s")

    @functools.partial(
        pl.kernel,
        mesh=mesh,
        out_type=jax.ShapeDtypeStruct((B, C, H, W), jnp.float32),
        scratch_types=[
            pltpu.VMEM((NROW * DROW,), jnp.float32),
            pltpu.VMEM((ROWS, H, W), jnp.float32),
            pltpu.SemaphoreType.DMA,
        ],
        compiler_params=pltpu.CompilerParams(needs_layout_passes=False),
    )
    def k(row_hbm, col_hbm, out_hbm, tab_v, chunk, sem):
        cid = lax.axis_index("c")
        sid = lax.axis_index("s")
        wid = sid * 2 + cid
        base_c = wid * ROWS
        is_col = base_c < D

        @pl.when(is_col)
        def _():
            pltpu.sync_copy(col_hbm, tab_v)

        @pl.when(jnp.logical_not(is_col))
        def _():
            pltpu.sync_copy(row_hbm, tab_v)

        iota = lax.iota(jnp.int32, _L)

        @pl.when(is_col)
        def _col_half():
            for r in range(ROWS):
                vecs = [
                    plsc.load_gather(
                        tab_v, [(iota + x0) * DCOL + (base_c + r)]
                    )
                    for x0 in range(0, W, _L)
                ]
                for y in range(H):
                    for i, v in enumerate(vecs):
                        chunk[r, y, pl.ds(i * _L, _L)] = v

        @pl.when(jnp.logical_not(is_col))
        def _row_half():
            for r in range(ROWS):
                ec = base_c - D + r
                for y in range(H):
                    v = plsc.load_gather(
                        tab_v, [jnp.full((_L,), y * DROW + ec, jnp.int32)]
                    )
                    for x0 in range(0, W, _L):
                        chunk[r, y, pl.ds(x0, _L)] = v

        copies = [
            pltpu.async_copy(
                chunk, out_hbm.at[b, pl.ds(base_c, ROWS)], sem
            )
            for b in range(B)
        ]
        for cp in copies:
            cp.wait()

    return k(row_embed.reshape(-1), col_embed.reshape(-1))


def kernel(x, row_embed, col_embed):
    B = x.shape[0]
    H, W = x.shape[-2], x.shape[-1]
    D = row_embed.shape[-1]
    return _pos_embed_sc(row_embed, col_embed, B, H, W, D)
